# Initial kernel scaffold; baseline (speedup 1.0000x reference)
import jax
import jax.numpy as jnp
from jax.experimental import pallas as pl


def kernel(m, edge_index, enc_mask_token, W_enc, b_enc, W_e2d, W_np, b_np, ln_g, ln_b, W_dec, b_dec):
    raise NotImplementedError("write your pallas kernel here")


# speedup vs baseline: 4.5764x; 4.5764x over previous
import functools

import jax
import jax.numpy as jnp
from jax import lax
from jax.experimental import pallas as pl
from jax.experimental.pallas import tpu as pltpu
from jax.experimental.pallas import tpu_sc as plsc

_N = 10000
_E = 320000
_D = 128
_NCLS = 16
_NMASK = 64000

_NC = 2
_NS = 16
_NW = _NC * _NS
_L = 16

_EPT = _E // _NW
_C = 80
_NCH = _EPT // _C
_MPT = _NMASK // _NW
_MCH = _MPT // _C
_NPAD = 10240

_mesh = plsc.VectorSubcoreMesh(core_axis_name="c", subcore_axis_name="s")


def _sc_encode_body(m_hbm, dstr_hbm, mask_hbm, mdst_hbm, tok_hbm,
                    accp_hbm, degp_hbm, xtrue_hbm,
                    dstbuf, rowbuf, maskbuf, mdstbuf,
                    tokbuf, onesbuf, zdegbuf, accsh, degsh, sem):
    c = lax.axis_index("c")
    s = lax.axis_index("s")
    w = s * _NC + c
    ebase = w * _EPT

    pltpu.sync_copy(dstr_hbm.at[w], dstbuf)
    pltpu.sync_copy(mask_hbm.at[w], maskbuf)
    pltpu.sync_copy(mdst_hbm.at[w], mdstbuf)
    pltpu.sync_copy(tok_hbm, tokbuf)

    ones16 = jnp.ones((_L,), jnp.float32)
    zero16 = jnp.zeros((_L,), jnp.float32)
    for i in range(_C // _L):
        onesbuf[pl.ds(i * _L, _L)] = ones16

    def _zdeg(i, carry):
        zdegbuf[pl.ds(i * _L, _L)] = zero16
        return carry
    lax.fori_loop(0, 640 // _L, _zdeg, 0)

    def _zrow(r, carry):
        for g in range(_D // _L):
            rowbuf[r, pl.ds(g * _L, _L)] = zero16
        return carry
    lax.fori_loop(0, _C, _zrow, 0)
    for z in range(8):
        pltpu.sync_copy(rowbuf, accsh.at[pl.ds(s * 640 + z * _C, _C)])
    pltpu.sync_copy(zdegbuf, degsh.at[pl.ds(s * 640, 640)])
    plsc.subcore_barrier()

    def _chunk(j, carry):
        pltpu.sync_copy(m_hbm.at[pl.ds(ebase + j * _C, _C)], rowbuf)
        pltpu.sync_copy(rowbuf, accsh.at[dstbuf.at[j]], add=True)
        pltpu.sync_copy(onesbuf, degsh.at[dstbuf.at[j]], add=True)
        return carry
    lax.fori_loop(0, _NCH, _chunk, 0)

    def _mchunk(j, carry):
        pltpu.async_copy(m_hbm.at[maskbuf.at[j]], rowbuf, sem).wait()
        pltpu.sync_copy(rowbuf, xtrue_hbm.at[pl.ds(w * _MPT + j * _C, _C)])
        def _crow(r, carry2):
            for g in range(_D // _L):
                sl = pl.ds(g * _L, _L)
                rowbuf[r, sl] = tokbuf[sl] - rowbuf[r, sl]
            return carry2
        lax.fori_loop(0, _C, _crow, 0)
        pltpu.sync_copy(rowbuf, accsh.at[mdstbuf.at[j]], add=True)
        return carry
    lax.fori_loop(0, _MCH, _mchunk, 0)

    plsc.subcore_barrier()

    pltpu.sync_copy(accsh.at[pl.ds(s * 640, 640)],
                    accp_hbm.at[c, pl.ds(s * 640, 640)])

    @pl.when(s == 0)
    def _deg_dump():
        pltpu.sync_copy(degsh, degp_hbm.at[pl.ds(c * _NPAD, _NPAD)])


def _sc_gather_scatter_body(h_hbm, srcr_hbm, dstr_hbm,
                            gp_hbm,
                            srcbuf, dstbuf, rowbuf, accsh, sem):
    c = lax.axis_index("c")
    s = lax.axis_index("s")
    w = s * _NC + c

    pltpu.sync_copy(srcr_hbm.at[w], srcbuf)
    pltpu.sync_copy(dstr_hbm.at[w], dstbuf)
    zero16 = jnp.zeros((_L,), jnp.float32)
    def _zrow(r, carry):
        for g in range(_D // _L):
            rowbuf[r, pl.ds(g * _L, _L)] = zero16
        return carry
    lax.fori_loop(0, _C, _zrow, 0)
    for z in range(8):
        pltpu.sync_copy(rowbuf, accsh.at[pl.ds(s * 640 + z * _C, _C)])
    plsc.subcore_barrier()

    def _chunk(j, carry):
        pltpu.async_copy(h_hbm.at[srcbuf.at[j]], rowbuf, sem).wait()
        pltpu.sync_copy(rowbuf, accsh.at[dstbuf.at[j]], add=True)
        return carry
    lax.fori_loop(0, _NCH, _chunk, 0)

    plsc.subcore_barrier()
    pltpu.sync_copy(accsh.at[pl.ds(s * 640, 640)],
                    gp_hbm.at[c, pl.ds(s * 640, 640)])


def _sc_edge_pred_body(h2_hbm, msrc_hbm, mdst_hbm,
                       xpred_hbm,
                       msbuf, mdbuf, abuf, bbuf, sem):
    c = lax.axis_index("c")
    s = lax.axis_index("s")
    w = s * _NC + c

    pltpu.sync_copy(msrc_hbm.at[w], msbuf)
    pltpu.sync_copy(mdst_hbm.at[w], mdbuf)

    def _chunk(j, carry):
        pltpu.async_copy(h2_hbm.at[msbuf.at[j]], abuf, sem).wait()
        pltpu.async_copy(h2_hbm.at[mdbuf.at[j]], bbuf, sem).wait()
        def _crow(r, carry2):
            for g in range(_D // _L):
                sl = pl.ds(g * _L, _L)
                abuf[r, sl] = 0.5 * (abuf[r, sl] + bbuf[r, sl])
            return carry2
        lax.fori_loop(0, _C, _crow, 0)
        pltpu.sync_copy(abuf, xpred_hbm.at[pl.ds(w * _MPT + j * _C, _C)])
        return carry
    lax.fori_loop(0, _MCH, _chunk, 0)


def _tc_encode_body(accp, degp, w_enc, b_enc, w_np, b_np, ln_g, ln_b,
                    h_out, ns_out):
    acc = accp[0] + accp[1]
    deg = degp[0] + degp[1]
    hin = acc / jnp.maximum(deg, 1.0)
    h = jnp.maximum(
        jnp.dot(hin, w_enc[...], preferred_element_type=jnp.float32)
        + b_enc[...], 0.0)
    h_out[...] = h
    lin = (jnp.dot(h, w_np[...], preferred_element_type=jnp.float32)
           + b_np[...])
    mu = jnp.mean(lin, axis=-1, keepdims=True)
    var = jnp.mean((lin - mu) ** 2, axis=-1, keepdims=True)
    ns_out[...] = (lin - mu) * lax.rsqrt(var + 1e-5) * ln_g[...] + ln_b[...]


def _tc_decode_body(gp, degp, h, w_e2d, w_dec, b_dec, h2_out):
    g = gp[0] + gp[1]
    deg = degp[0] + degp[1]
    t = 0.5 * (g + deg * h[...]) / jnp.maximum(deg, 1.0)
    u = jnp.dot(t, w_e2d[...], preferred_element_type=jnp.float32)
    h2_out[...] = jnp.maximum(
        jnp.dot(u, w_dec[...], preferred_element_type=jnp.float32)
        + b_dec[...], 0.0)


_sc_encode = pl.kernel(
    _sc_encode_body,
    out_type=(
        jax.ShapeDtypeStruct((_NC, _NPAD, _D), jnp.float32),
        jax.ShapeDtypeStruct((_NC * _NPAD,), jnp.float32),
        jax.ShapeDtypeStruct((_NMASK, _D), jnp.float32),
    ),
    mesh=_mesh,
    scratch_types=[
        pltpu.VMEM((_NCH, _C), jnp.int32),
        pltpu.VMEM((_C, _D), jnp.float32),
        pltpu.VMEM((_MCH, _C), jnp.int32),
        pltpu.VMEM((_MCH, _C), jnp.int32),
        pltpu.VMEM((_D,), jnp.float32),
        pltpu.VMEM((_C,), jnp.float32),
        pltpu.VMEM((640,), jnp.float32),
        pltpu.VMEM_SHARED((_NPAD, _D), jnp.float32),
        pltpu.VMEM_SHARED((_NPAD,), jnp.float32),
        pltpu.SemaphoreType.DMA,
    ],
)

_sc_gather_scatter = pl.kernel(
    _sc_gather_scatter_body,
    out_type=jax.ShapeDtypeStruct((_NC, _NPAD, _D), jnp.float32),
    mesh=_mesh,
    scratch_types=[
        pltpu.VMEM((_NCH, _C), jnp.int32),
        pltpu.VMEM((_NCH, _C), jnp.int32),
        pltpu.VMEM((_C, _D), jnp.float32),
        pltpu.VMEM_SHARED((_NPAD, _D), jnp.float32),
        pltpu.SemaphoreType.DMA,
    ],
)

_sc_edge_pred = pl.kernel(
    _sc_edge_pred_body,
    out_type=jax.ShapeDtypeStruct((_NMASK, _D), jnp.float32),
    mesh=_mesh,
    scratch_types=[
        pltpu.VMEM((_MCH, _C), jnp.int32),
        pltpu.VMEM((_MCH, _C), jnp.int32),
        pltpu.VMEM((_C, _D), jnp.float32),
        pltpu.VMEM((_C, _D), jnp.float32),
        pltpu.SemaphoreType.DMA,
    ],
)

_tc_encode = pl.pallas_call(
    _tc_encode_body,
    out_shape=(
        jax.ShapeDtypeStruct((_NPAD, _D), jnp.float32),
        jax.ShapeDtypeStruct((_NPAD, _NCLS), jnp.float32),
    ),
)

_tc_decode = pl.pallas_call(
    _tc_decode_body,
    out_shape=jax.ShapeDtypeStruct((_NPAD, _D), jnp.float32),
)


def kernel(m, edge_index, enc_mask_token, W_enc, b_enc, W_e2d, W_np, b_np,
           ln_g, ln_b, W_dec, b_dec):
    src = edge_index[0]
    dst = edge_index[1]
    perm = jax.random.permutation(jax.random.key(42), _E)
    mask_edges = perm[:_NMASK].astype(jnp.int32)
    msrc = jnp.take(src, mask_edges)
    mdst = jnp.take(dst, mask_edges)

    src_r = src.reshape(_NW, _NCH, _C)
    dst_r = dst.reshape(_NW, _NCH, _C)
    mask_r = mask_edges.reshape(_NW, _MCH, _C)
    msrc_r = msrc.reshape(_NW, _MCH, _C)
    mdst_r = mdst.reshape(_NW, _MCH, _C)
    accp, degp, x_true = _sc_encode(m, dst_r, mask_r, mdst_r,
                                    enc_mask_token[0])
    degp2 = degp.reshape(_NC, _NPAD, 1)
    h, ns = _tc_encode(accp, degp2, W_enc, b_enc.reshape(1, _D), W_np,
                       b_np.reshape(1, _NCLS), ln_g.reshape(1, _NCLS),
                       ln_b.reshape(1, _NCLS))
    gp = _sc_gather_scatter(h, src_r, dst_r)
    h2 = _tc_decode(gp, degp2, h, W_e2d, W_dec, b_dec.reshape(1, _D))
    x_pred = _sc_edge_pred(h2, msrc_r, mdst_r)

    return (x_pred, x_true, ns[:_N])


# speedup vs baseline: 9.2765x; 2.0270x over previous
import functools

import jax
import jax.numpy as jnp
from jax import lax
from jax.experimental import pallas as pl
from jax.experimental.pallas import tpu as pltpu
from jax.experimental.pallas import tpu_sc as plsc

_N = 10000
_E = 320000
_D = 128
_NCLS = 16
_NMASK = 64000

_NC = 2
_NS = 16
_NW = _NC * _NS
_L = 16

_EPT = _E // _NW
_C = 80
_NCH = _EPT // _C
_MPT = _NMASK // _NW
_MCH = _MPT // _C
_NPAD = 10240

_mesh = plsc.VectorSubcoreMesh(core_axis_name="c", subcore_axis_name="s")

_MASK_CACHE = []


def _mask_edges_const():
    if not _MASK_CACHE:
        with jax.ensure_compile_time_eval():
            perm = jax.random.permutation(jax.random.key(42), _E)
            _MASK_CACHE.append(jnp.asarray(perm[:_NMASK], jnp.int32))
    return _MASK_CACHE[0]


def _sc_encode_body(m_hbm, dstr_hbm, mask_hbm, mdst_hbm, tok_hbm,
                    accp_hbm, degp_hbm, xtrue_hbm,
                    dstbuf, rowbuf, maskbuf, mdstbuf,
                    tokbuf, onesbuf, zdegbuf, accsh, degsh, sem):
    c = lax.axis_index("c")
    s = lax.axis_index("s")
    w = s * _NC + c
    ebase = w * _EPT

    pltpu.sync_copy(dstr_hbm.at[w], dstbuf)
    pltpu.sync_copy(mask_hbm.at[w], maskbuf)
    pltpu.sync_copy(mdst_hbm.at[w], mdstbuf)
    pltpu.sync_copy(tok_hbm, tokbuf)

    ones16 = jnp.ones((_L,), jnp.float32)
    zero16 = jnp.zeros((_L,), jnp.float32)
    for i in range(_C // _L):
        onesbuf[pl.ds(i * _L, _L)] = ones16

    def _zdeg(i, carry):
        zdegbuf[pl.ds(i * _L, _L)] = zero16
        return carry
    lax.fori_loop(0, 640 // _L, _zdeg, 0)

    def _zrow(r, carry):
        for g in range(_D // _L):
            rowbuf[r, pl.ds(g * _L, _L)] = zero16
        return carry
    lax.fori_loop(0, _C, _zrow, 0)
    for z in range(8):
        pltpu.sync_copy(rowbuf, accsh.at[pl.ds(s * 640 + z * _C, _C)])
    pltpu.sync_copy(zdegbuf, degsh.at[pl.ds(s * 640, 640)])
    plsc.subcore_barrier()

    def _chunk(j, carry):
        pltpu.sync_copy(m_hbm.at[pl.ds(ebase + j * _C, _C)], rowbuf)
        pltpu.sync_copy(rowbuf, accsh.at[dstbuf.at[j]], add=True)
        pltpu.sync_copy(onesbuf, degsh.at[dstbuf.at[j]], add=True)
        return carry
    lax.fori_loop(0, _NCH, _chunk, 0)

    def _mchunk(j, carry):
        pltpu.async_copy(m_hbm.at[maskbuf.at[j]], rowbuf, sem).wait()
        pltpu.sync_copy(rowbuf, xtrue_hbm.at[pl.ds(w * _MPT + j * _C, _C)])
        def _crow(r, carry2):
            for g in range(_D // _L):
                sl = pl.ds(g * _L, _L)
                rowbuf[r, sl] = tokbuf[sl] - rowbuf[r, sl]
            return carry2
        lax.fori_loop(0, _C, _crow, 0)
        pltpu.sync_copy(rowbuf, accsh.at[mdstbuf.at[j]], add=True)
        return carry
    lax.fori_loop(0, _MCH, _mchunk, 0)

    plsc.subcore_barrier()

    pltpu.sync_copy(accsh.at[pl.ds(s * 640, 640)],
                    accp_hbm.at[c, pl.ds(s * 640, 640)])

    @pl.when(s == 0)
    def _deg_dump():
        pltpu.sync_copy(degsh, degp_hbm.at[pl.ds(c * _NPAD, _NPAD)])


def _sc_gather_scatter_body(h_hbm, srcr_hbm, dstr_hbm,
                            gp_hbm,
                            srcbuf, dstbuf, rowbuf, accsh, sem):
    c = lax.axis_index("c")
    s = lax.axis_index("s")
    w = s * _NC + c

    pltpu.sync_copy(srcr_hbm.at[w], srcbuf)
    pltpu.sync_copy(dstr_hbm.at[w], dstbuf)
    zero16 = jnp.zeros((_L,), jnp.float32)
    def _zrow(r, carry):
        for g in range(_D // _L):
            rowbuf[r, pl.ds(g * _L, _L)] = zero16
        return carry
    lax.fori_loop(0, _C, _zrow, 0)
    for z in range(8):
        pltpu.sync_copy(rowbuf, accsh.at[pl.ds(s * 640 + z * _C, _C)])
    plsc.subcore_barrier()

    def _chunk(j, carry):
        pltpu.async_copy(h_hbm.at[srcbuf.at[j]], rowbuf, sem).wait()
        pltpu.sync_copy(rowbuf, accsh.at[dstbuf.at[j]], add=True)
        return carry
    lax.fori_loop(0, _NCH, _chunk, 0)

    plsc.subcore_barrier()
    pltpu.sync_copy(accsh.at[pl.ds(s * 640, 640)],
                    gp_hbm.at[c, pl.ds(s * 640, 640)])


def _sc_edge_pred_body(h2_hbm, msrc_hbm, mdst_hbm,
                       xpred_hbm,
                       msbuf, mdbuf, abuf, bbuf, sem):
    c = lax.axis_index("c")
    s = lax.axis_index("s")
    w = s * _NC + c

    pltpu.sync_copy(msrc_hbm.at[w], msbuf)
    pltpu.sync_copy(mdst_hbm.at[w], mdbuf)

    def _chunk(j, carry):
        pltpu.async_copy(h2_hbm.at[msbuf.at[j]], abuf, sem).wait()
        pltpu.async_copy(h2_hbm.at[mdbuf.at[j]], bbuf, sem).wait()
        def _crow(r, carry2):
            for g in range(_D // _L):
                sl = pl.ds(g * _L, _L)
                abuf[r, sl] = 0.5 * (abuf[r, sl] + bbuf[r, sl])
            return carry2
        lax.fori_loop(0, _C, _crow, 0)
        pltpu.sync_copy(abuf, xpred_hbm.at[pl.ds(w * _MPT + j * _C, _C)])
        return carry
    lax.fori_loop(0, _MCH, _chunk, 0)


def _tc_encode_body(accp, degp, w_enc, b_enc, w_np, b_np, ln_g, ln_b,
                    h_out, ns_out):
    acc = accp[0] + accp[1]
    deg = degp[0] + degp[1]
    hin = acc / jnp.maximum(deg, 1.0)
    h = jnp.maximum(
        jnp.dot(hin, w_enc[...], preferred_element_type=jnp.float32)
        + b_enc[...], 0.0)
    h_out[...] = h
    lin = (jnp.dot(h, w_np[...], preferred_element_type=jnp.float32)
           + b_np[...])
    mu = jnp.mean(lin, axis=-1, keepdims=True)
    var = jnp.mean((lin - mu) ** 2, axis=-1, keepdims=True)
    ns_out[...] = (lin - mu) * lax.rsqrt(var + 1e-5) * ln_g[...] + ln_b[...]


def _tc_decode_body(gp, degp, h, w_e2d, w_dec, b_dec, h2_out):
    g = gp[0] + gp[1]
    deg = degp[0] + degp[1]
    t = 0.5 * (g + deg * h[...]) / jnp.maximum(deg, 1.0)
    u = jnp.dot(t, w_e2d[...], preferred_element_type=jnp.float32)
    h2_out[...] = jnp.maximum(
        jnp.dot(u, w_dec[...], preferred_element_type=jnp.float32)
        + b_dec[...], 0.0)


_sc_encode = pl.kernel(
    _sc_encode_body,
    out_type=(
        jax.ShapeDtypeStruct((_NC, _NPAD, _D), jnp.float32),
        jax.ShapeDtypeStruct((_NC * _NPAD,), jnp.float32),
        jax.ShapeDtypeStruct((_NMASK, _D), jnp.float32),
    ),
    mesh=_mesh,
    scratch_types=[
        pltpu.VMEM((_NCH, _C), jnp.int32),
        pltpu.VMEM((_C, _D), jnp.float32),
        pltpu.VMEM((_MCH, _C), jnp.int32),
        pltpu.VMEM((_MCH, _C), jnp.int32),
        pltpu.VMEM((_D,), jnp.float32),
        pltpu.VMEM((_C,), jnp.float32),
        pltpu.VMEM((640,), jnp.float32),
        pltpu.VMEM_SHARED((_NPAD, _D), jnp.float32),
        pltpu.VMEM_SHARED((_NPAD,), jnp.float32),
        pltpu.SemaphoreType.DMA,
    ],
)

_sc_gather_scatter = pl.kernel(
    _sc_gather_scatter_body,
    out_type=jax.ShapeDtypeStruct((_NC, _NPAD, _D), jnp.float32),
    mesh=_mesh,
    scratch_types=[
        pltpu.VMEM((_NCH, _C), jnp.int32),
        pltpu.VMEM((_NCH, _C), jnp.int32),
        pltpu.VMEM((_C, _D), jnp.float32),
        pltpu.VMEM_SHARED((_NPAD, _D), jnp.float32),
        pltpu.SemaphoreType.DMA,
    ],
)

_sc_edge_pred = pl.kernel(
    _sc_edge_pred_body,
    out_type=jax.ShapeDtypeStruct((_NMASK, _D), jnp.float32),
    mesh=_mesh,
    scratch_types=[
        pltpu.VMEM((_MCH, _C), jnp.int32),
        pltpu.VMEM((_MCH, _C), jnp.int32),
        pltpu.VMEM((_C, _D), jnp.float32),
        pltpu.VMEM((_C, _D), jnp.float32),
        pltpu.SemaphoreType.DMA,
    ],
)

_tc_encode = pl.pallas_call(
    _tc_encode_body,
    out_shape=(
        jax.ShapeDtypeStruct((_NPAD, _D), jnp.float32),
        jax.ShapeDtypeStruct((_NPAD, _NCLS), jnp.float32),
    ),
)

_tc_decode = pl.pallas_call(
    _tc_decode_body,
    out_shape=jax.ShapeDtypeStruct((_NPAD, _D), jnp.float32),
)


def kernel(m, edge_index, enc_mask_token, W_enc, b_enc, W_e2d, W_np, b_np,
           ln_g, ln_b, W_dec, b_dec):
    src = edge_index[0]
    dst = edge_index[1]
    mask_edges = _mask_edges_const()
    msrc = jnp.take(src, mask_edges)
    mdst = jnp.take(dst, mask_edges)

    src_r = src.reshape(_NW, _NCH, _C)
    dst_r = dst.reshape(_NW, _NCH, _C)
    mask_r = mask_edges.reshape(_NW, _MCH, _C)
    msrc_r = msrc.reshape(_NW, _MCH, _C)
    mdst_r = mdst.reshape(_NW, _MCH, _C)
    accp, degp, x_true = _sc_encode(m, dst_r, mask_r, mdst_r,
                                    enc_mask_token[0])
    degp2 = degp.reshape(_NC, _NPAD, 1)
    h, ns = _tc_encode(accp, degp2, W_enc, b_enc.reshape(1, _D), W_np,
                       b_np.reshape(1, _NCLS), ln_g.reshape(1, _NCLS),
                       ln_b.reshape(1, _NCLS))
    gp = _sc_gather_scatter(h, src_r, dst_r)
    h2 = _tc_decode(gp, degp2, h, W_e2d, W_dec, b_dec.reshape(1, _D))
    x_pred = _sc_edge_pred(h2, msrc_r, mdst_r)

    return (x_pred, x_true, ns[:_N])


# speedup vs baseline: 11.9839x; 1.2919x over previous
import functools

import jax
import jax.numpy as jnp
from jax import lax
from jax.experimental import pallas as pl
from jax.experimental.pallas import tpu as pltpu
from jax.experimental.pallas import tpu_sc as plsc

_N = 10000
_E = 320000
_D = 128
_NCLS = 16
_NMASK = 64000

_NC = 2
_NS = 16
_NW = _NC * _NS
_L = 16

_EPT = _E // _NW
_C = 80
_NCH = _EPT // _C
_C2 = 40
_NCH2 = _EPT // _C2
_MPT = _NMASK // _NW
_MCH = _MPT // _C
_NPAD = 10240

_mesh = plsc.VectorSubcoreMesh(core_axis_name="c", subcore_axis_name="s")

_MASK_CACHE = []


def _mask_edges_const():
    if not _MASK_CACHE:
        with jax.ensure_compile_time_eval():
            perm = jax.random.permutation(jax.random.key(42), _E)
            _MASK_CACHE.append(jnp.asarray(perm[:_NMASK], jnp.int32))
    return _MASK_CACHE[0]


def _sc_encode_body(m_hbm, dstr_hbm, mask_hbm, mdst_hbm, tok_hbm,
                    accp_hbm, degp_hbm, xtrue_hbm,
                    dstbuf, rowbuf, rowbuf2, maskbuf, mdstbuf,
                    tokbuf, onesbuf, zdegbuf, accsh, degsh, sem, sem2):
    c = lax.axis_index("c")
    s = lax.axis_index("s")
    w = s * _NC + c
    ebase = w * _EPT

    pltpu.sync_copy(dstr_hbm.at[w], dstbuf)
    pltpu.sync_copy(mask_hbm.at[w], maskbuf)
    pltpu.sync_copy(mdst_hbm.at[w], mdstbuf)
    pltpu.sync_copy(tok_hbm, tokbuf)

    ones16 = jnp.ones((_L,), jnp.float32)
    zero16 = jnp.zeros((_L,), jnp.float32)
    for i in range(_C // _L):
        onesbuf[pl.ds(i * _L, _L)] = ones16

    def _zdeg(i, carry):
        zdegbuf[pl.ds(i * _L, _L)] = zero16
        return carry
    lax.fori_loop(0, 640 // _L, _zdeg, 0)

    def _zrow(r, carry):
        for g in range(_D // _L):
            rowbuf[r, pl.ds(g * _L, _L)] = zero16
        return carry
    lax.fori_loop(0, _C, _zrow, 0)
    for z in range(8):
        pltpu.sync_copy(rowbuf, accsh.at[pl.ds(s * 640 + z * _C, _C)])
    pltpu.sync_copy(zdegbuf, degsh.at[pl.ds(s * 640, 640)])
    plsc.subcore_barrier()

    bufs = (rowbuf, rowbuf2)
    sems = (sem, sem2)

    def _mstart(j, b):
        pltpu.async_copy(m_hbm.at[pl.ds(ebase + j * _C, _C)], bufs[b], sems[b])

    def _mwait(j, b):
        pltpu.make_async_copy(m_hbm.at[pl.ds(ebase + j * _C, _C)], bufs[b],
                              sems[b]).wait()

    def _drain(j, b):
        pltpu.sync_copy(bufs[b], accsh.at[dstbuf.at[j]], add=True)
        pltpu.sync_copy(onesbuf, degsh.at[dstbuf.at[j]], add=True)

    _mstart(0, 0)
    def _pair(p, carry):
        j0 = 2 * p
        _mstart(j0 + 1, 1)
        _mwait(j0, 0)
        _drain(j0, 0)
        @pl.when(j0 + 2 < _NCH)
        def _():
            _mstart(j0 + 2, 0)
        _mwait(j0 + 1, 1)
        _drain(j0 + 1, 1)
        return carry
    lax.fori_loop(0, (_NCH - 1) // 2, _pair, 0)
    _mwait(_NCH - 1, 0)
    _drain(_NCH - 1, 0)

    def _gstart(j, b):
        pltpu.async_copy(m_hbm.at[maskbuf.at[j]], bufs[b], sems[b])

    def _gwait(j, b):
        pltpu.make_async_copy(m_hbm.at[maskbuf.at[j]], bufs[b], sems[b]).wait()

    def _mdrain(j, b):
        buf = bufs[b]
        pltpu.sync_copy(buf, xtrue_hbm.at[pl.ds(w * _MPT + j * _C, _C)])
        def _crow(r, carry2):
            for g in range(_D // _L):
                sl = pl.ds(g * _L, _L)
                buf[r, sl] = tokbuf[sl] - buf[r, sl]
            return carry2
        lax.fori_loop(0, _C, _crow, 0)
        pltpu.sync_copy(buf, accsh.at[mdstbuf.at[j]], add=True)

    _gstart(0, 0)
    def _mpair(p, carry):
        j0 = 2 * p
        _gstart(j0 + 1, 1)
        _gwait(j0, 0)
        _mdrain(j0, 0)
        @pl.when(j0 + 2 < _MCH)
        def _():
            _gstart(j0 + 2, 0)
        _gwait(j0 + 1, 1)
        _mdrain(j0 + 1, 1)
        return carry
    lax.fori_loop(0, (_MCH - 1) // 2, _mpair, 0)
    _gwait(_MCH - 1, 0)
    _mdrain(_MCH - 1, 0)

    plsc.subcore_barrier()

    pltpu.sync_copy(accsh.at[pl.ds(s * 640, 640)],
                    accp_hbm.at[c, pl.ds(s * 640, 640)])

    @pl.when(s == 0)
    def _deg_dump():
        pltpu.sync_copy(degsh, degp_hbm.at[pl.ds(c * _NPAD, _NPAD)])


def _sc_gather_scatter_body(h_hbm, srcr_hbm, dstr_hbm,
                            gp_hbm,
                            srcbuf, dstbuf, rowbuf, accsh, sem):
    c = lax.axis_index("c")
    s = lax.axis_index("s")
    w = s * _NC + c

    pltpu.sync_copy(srcr_hbm.at[w], srcbuf)
    pltpu.sync_copy(dstr_hbm.at[w], dstbuf)
    zero16 = jnp.zeros((_L,), jnp.float32)
    def _zrow(r, carry):
        for g in range(_D // _L):
            rowbuf[r, pl.ds(g * _L, _L)] = zero16
        return carry
    lax.fori_loop(0, _C, _zrow, 0)
    for z in range(8):
        pltpu.sync_copy(rowbuf, accsh.at[pl.ds(s * 640 + z * _C, _C)])
    plsc.subcore_barrier()

    def _chunk(j, carry):
        pltpu.async_copy(h_hbm.at[srcbuf.at[j]], rowbuf, sem).wait()
        pltpu.sync_copy(rowbuf, accsh.at[dstbuf.at[j]], add=True)
        return carry
    lax.fori_loop(0, _NCH, _chunk, 0)

    plsc.subcore_barrier()
    pltpu.sync_copy(accsh.at[pl.ds(s * 640, 640)],
                    gp_hbm.at[c, pl.ds(s * 640, 640)])


def _sc_edge_pred_body(h2_hbm, msrc_hbm, mdst_hbm,
                       xpred_hbm,
                       msbuf, mdbuf, abuf, bbuf, abuf2, bbuf2,
                       semA, semB, semA2, semB2):
    c = lax.axis_index("c")
    s = lax.axis_index("s")
    w = s * _NC + c

    pltpu.sync_copy(msrc_hbm.at[w], msbuf)
    pltpu.sync_copy(mdst_hbm.at[w], mdbuf)

    abufs = (abuf, abuf2)
    bbufs = (bbuf, bbuf2)
    semsA = (semA, semA2)
    semsB = (semB, semB2)

    def _gstart(j, b):
        pltpu.async_copy(h2_hbm.at[msbuf.at[j]], abufs[b], semsA[b])
        pltpu.async_copy(h2_hbm.at[mdbuf.at[j]], bbufs[b], semsB[b])

    def _gwait(j, b):
        pltpu.make_async_copy(h2_hbm.at[msbuf.at[j]], abufs[b],
                              semsA[b]).wait()
        pltpu.make_async_copy(h2_hbm.at[mdbuf.at[j]], bbufs[b],
                              semsB[b]).wait()

    def _drain(j, b):
        ab, bb = abufs[b], bbufs[b]
        def _crow(r, carry2):
            for g in range(_D // _L):
                sl = pl.ds(g * _L, _L)
                ab[r, sl] = 0.5 * (ab[r, sl] + bb[r, sl])
            return carry2
        lax.fori_loop(0, _C, _crow, 0)
        pltpu.sync_copy(ab, xpred_hbm.at[pl.ds(w * _MPT + j * _C, _C)])

    _gstart(0, 0)
    def _pair(p, carry):
        j0 = 2 * p
        _gstart(j0 + 1, 1)
        _gwait(j0, 0)
        _drain(j0, 0)
        @pl.when(j0 + 2 < _MCH)
        def _():
            _gstart(j0 + 2, 0)
        _gwait(j0 + 1, 1)
        _drain(j0 + 1, 1)
        return carry
    lax.fori_loop(0, (_MCH - 1) // 2, _pair, 0)
    _gwait(_MCH - 1, 0)
    _drain(_MCH - 1, 0)


def _tc_encode_body(accp, degp, w_enc, b_enc, w_np, b_np, ln_g, ln_b,
                    h_out, ns_out):
    acc = accp[0] + accp[1]
    deg = degp[0] + degp[1]
    hin = acc / jnp.maximum(deg, 1.0)
    h = jnp.maximum(
        jnp.dot(hin, w_enc[...], preferred_element_type=jnp.float32)
        + b_enc[...], 0.0)
    h_out[...] = h
    lin = (jnp.dot(h, w_np[...], preferred_element_type=jnp.float32)
           + b_np[...])
    mu = jnp.mean(lin, axis=-1, keepdims=True)
    var = jnp.mean((lin - mu) ** 2, axis=-1, keepdims=True)
    ns_out[...] = (lin - mu) * lax.rsqrt(var + 1e-5) * ln_g[...] + ln_b[...]


def _tc_decode_body(gp, degp, h, w_e2d, w_dec, b_dec, h2_out):
    g = gp[0] + gp[1]
    deg = degp[0] + degp[1]
    t = 0.5 * (g + deg * h[...]) / jnp.maximum(deg, 1.0)
    u = jnp.dot(t, w_e2d[...], preferred_element_type=jnp.float32)
    h2_out[...] = jnp.maximum(
        jnp.dot(u, w_dec[...], preferred_element_type=jnp.float32)
        + b_dec[...], 0.0)


_sc_encode = pl.kernel(
    _sc_encode_body,
    out_type=(
        jax.ShapeDtypeStruct((_NC, _NPAD, _D), jnp.float32),
        jax.ShapeDtypeStruct((_NC * _NPAD,), jnp.float32),
        jax.ShapeDtypeStruct((_NMASK, _D), jnp.float32),
    ),
    mesh=_mesh,
    scratch_types=[
        pltpu.VMEM((_NCH, _C), jnp.int32),
        pltpu.VMEM((_C, _D), jnp.float32),
        pltpu.VMEM((_C, _D), jnp.float32),
        pltpu.VMEM((_MCH, _C), jnp.int32),
        pltpu.VMEM((_MCH, _C), jnp.int32),
        pltpu.VMEM((_D,), jnp.float32),
        pltpu.VMEM((_C,), jnp.float32),
        pltpu.VMEM((640,), jnp.float32),
        pltpu.VMEM_SHARED((_NPAD, _D), jnp.float32),
        pltpu.VMEM_SHARED((_NPAD,), jnp.float32),
        pltpu.SemaphoreType.DMA,
        pltpu.SemaphoreType.DMA,
    ],
)

_sc_gather_scatter = pl.kernel(
    _sc_gather_scatter_body,
    out_type=jax.ShapeDtypeStruct((_NC, _NPAD, _D), jnp.float32),
    mesh=_mesh,
    scratch_types=[
        pltpu.VMEM((_NCH, _C), jnp.int32),
        pltpu.VMEM((_NCH, _C), jnp.int32),
        pltpu.VMEM((_C, _D), jnp.float32),
        pltpu.VMEM_SHARED((_NPAD, _D), jnp.float32),
        pltpu.SemaphoreType.DMA,
    ],
)

_sc_edge_pred = pl.kernel(
    _sc_edge_pred_body,
    out_type=jax.ShapeDtypeStruct((_NMASK, _D), jnp.float32),
    mesh=_mesh,
    scratch_types=[
        pltpu.VMEM((_MCH, _C), jnp.int32),
        pltpu.VMEM((_MCH, _C), jnp.int32),
        pltpu.VMEM((_C, _D), jnp.float32),
        pltpu.VMEM((_C, _D), jnp.float32),
        pltpu.VMEM((_C, _D), jnp.float32),
        pltpu.VMEM((_C, _D), jnp.float32),
        pltpu.SemaphoreType.DMA,
        pltpu.SemaphoreType.DMA,
        pltpu.SemaphoreType.DMA,
        pltpu.SemaphoreType.DMA,
    ],
)

_tc_encode = pl.pallas_call(
    _tc_encode_body,
    out_shape=(
        jax.ShapeDtypeStruct((_NPAD, _D), jnp.float32),
        jax.ShapeDtypeStruct((_NPAD, _NCLS), jnp.float32),
    ),
)

_tc_decode = pl.pallas_call(
    _tc_decode_body,
    out_shape=jax.ShapeDtypeStruct((_NPAD, _D), jnp.float32),
)


def kernel(m, edge_index, enc_mask_token, W_enc, b_enc, W_e2d, W_np, b_np,
           ln_g, ln_b, W_dec, b_dec):
    src = edge_index[0]
    dst = edge_index[1]
    mask_edges = _mask_edges_const()
    msrc = jnp.take(src, mask_edges)
    mdst = jnp.take(dst, mask_edges)

    src_r = src.reshape(_NW, _NCH, _C)
    dst_r = dst.reshape(_NW, _NCH, _C)
    mask_r = mask_edges.reshape(_NW, _MCH, _C)
    msrc_r = msrc.reshape(_NW, _MCH, _C)
    mdst_r = mdst.reshape(_NW, _MCH, _C)
    accp, degp, x_true = _sc_encode(m, dst_r, mask_r, mdst_r,
                                    enc_mask_token[0])
    degp2 = degp.reshape(_NC, _NPAD, 1)
    h, ns = _tc_encode(accp, degp2, W_enc, b_enc.reshape(1, _D), W_np,
                       b_np.reshape(1, _NCLS), ln_g.reshape(1, _NCLS),
                       ln_b.reshape(1, _NCLS))
    gp = _sc_gather_scatter(h, src_r, dst_r)
    h2 = _tc_decode(gp, degp2, h, W_e2d, W_dec, b_dec.reshape(1, _D))
    x_pred = _sc_edge_pred(h2, msrc_r, mdst_r)

    return (x_pred, x_true, ns[:_N])
